# baseline (speedup 1.0000x reference)
import functools

import jax
import jax.numpy as jnp
from jax import lax
from jax.experimental import pallas as pl
from jax.experimental.pallas import tpu as pltpu
from jax.experimental.pallas import tpu_sc as plsc

NC = 2
NS = 16
NW = NC * NS
CHUNK = 32


def _make_sc_kernel(seq_len, b_sz, emb, dtype):
    rows_per_w = seq_len // NW
    n_chunks = rows_per_w // CHUNK
    mesh = plsc.VectorSubcoreMesh(core_axis_name="c", subcore_axis_name="s")

    @functools.partial(
        pl.kernel,
        out_type=jax.ShapeDtypeStruct((seq_len, b_sz, emb), dtype),
        mesh=mesh,
        scratch_types=[
            pltpu.VMEM((CHUNK, emb), dtype),
            pltpu.SemaphoreType.DMA,
        ],
    )
    def sc_kernel(w_hbm, out_hbm, buf, sem):
        wid = lax.axis_index("s") * NC + lax.axis_index("c")
        base = wid * rows_per_w
        for c in range(n_chunks):
            row0 = base + c * CHUNK
            pltpu.sync_copy(w_hbm.at[pl.ds(row0, CHUNK)], buf)
            for b in range(b_sz):
                pltpu.sync_copy(buf, out_hbm.at[pl.ds(row0, CHUNK), b])

    return sc_kernel


def kernel(inputs, W):
    seq_len, b_sz = inputs.shape
    emb = W.shape[1]
    return _make_sc_kernel(seq_len, b_sz, emb, W.dtype)(W[:seq_len])


# speedup vs baseline: 1.0463x; 1.0463x over previous
import functools

import jax
import jax.numpy as jnp
from jax import lax
from jax.experimental import pallas as pl
from jax.experimental.pallas import tpu as pltpu
from jax.experimental.pallas import tpu_sc as plsc

NC = 2
NS = 16
NW = NC * NS
CHUNK = 32


NBUF = 2


def _make_sc_kernel(seq_len, b_sz, emb, dtype):
    rows_per_w = seq_len // NW
    n_chunks = rows_per_w // CHUNK
    mesh = plsc.VectorSubcoreMesh(core_axis_name="c", subcore_axis_name="s")

    @functools.partial(
        pl.kernel,
        out_type=jax.ShapeDtypeStruct((seq_len, b_sz, emb), dtype),
        mesh=mesh,
        scratch_types=[
            pltpu.VMEM((NBUF, CHUNK, emb), dtype),
            pltpu.SemaphoreType.DMA,
            pltpu.SemaphoreType.DMA,
        ],
    )
    def sc_kernel(w_hbm, out_hbm, buf, rsem, wsem):
        wid = lax.axis_index("s") * NC + lax.axis_index("c")
        base = wid * rows_per_w

        def read(c):
            return pltpu.async_copy(
                w_hbm.at[pl.ds(base + c * CHUNK, CHUNK)], buf.at[c % NBUF], rsem
            )

        def writes(c):
            return [
                pltpu.async_copy(
                    buf.at[c % NBUF],
                    out_hbm.at[pl.ds(base + c * CHUNK, CHUNK), b],
                    wsem,
                )
                for b in range(b_sz)
            ]

        rds = {c: None for c in range(n_chunks)}
        wrs = {}
        rds[0] = read(0)
        if n_chunks > 1:
            rds[1] = read(1)
        for c in range(n_chunks):
            rds[c].wait()
            wrs[c] = writes(c)
            if c + NBUF < n_chunks:
                for d in wrs[c]:
                    d.wait()
                rds[c + NBUF] = read(c + NBUF)
        for c in range(max(0, n_chunks - NBUF), n_chunks):
            for d in wrs[c]:
                d.wait()

    return sc_kernel


def kernel(inputs, W):
    seq_len, b_sz = inputs.shape
    emb = W.shape[1]
    return _make_sc_kernel(seq_len, b_sz, emb, W.dtype)(W[:seq_len])
